# baseline (speedup 1.0000x reference)
import functools

import jax
import jax.numpy as jnp
from jax import lax
from jax.experimental import pallas as pl
from jax.experimental.pallas import tpu as pltpu
from jax.experimental.pallas import tpu_sc as plsc

NUM_PARTS = 8
POSE_NUM = 16
NC = 48
G = 512
PD = 20
H = 128
MAT_MODE = ((0, 1), (0, 2), (1, 2))
NW = 32
CH = 128


def _pose_select_body(ap_ref, qp_ref, pid_ref, wscale_ref):
    ap = ap_ref[...]
    qp = qp_ref[...]

    def quat(aa):
        angle = jnp.sqrt(jnp.sum(aa * aa, axis=-1))
        half = 0.5 * angle
        small = angle < 1e-6
        sinc = jnp.where(small, 0.5 - angle * angle / 48.0,
                         jnp.sin(half) / jnp.where(small, 1.0, angle))
        return jnp.cos(half), aa * sinc[..., None]

    kw, kxyz = quat(ap)
    qw, qxyz = quat(qp)
    dot = kw * qw + jnp.sum(kxyz * qxyz, axis=-1)
    absdot = jnp.abs(dot)
    iota = jax.lax.broadcasted_iota(jnp.int32, (POSE_NUM, 1), 0)
    for p in range(NUM_PARTS):
        pd = jnp.sum(absdot[:, 3 * p:3 * p + 3], axis=-1, keepdims=True)
        maxv = jnp.max(pd)
        pid = jnp.min(jnp.where(pd == maxv, iota, POSE_NUM))
        pid_ref[p] = pid.astype(jnp.int32)
        wscale_ref[p] = maxv / jnp.maximum(maxv, 1e-16)


def _sc_gather_body(tab_ref, tpts_ref, pa_hbm, pm_hbm, prb_hbm, out_ref,
                    coords_v, idx_v, rows_v, pa_v, pm_v, prb_v, sem):
    n = tpts_ref.shape[1]
    npt = n // NW
    nchunk = npt // CH
    wid = lax.axis_index("s") * 2 + lax.axis_index("c")
    n0 = wid * npt
    pltpu.sync_copy(pa_hbm, pa_v)
    pltpu.sync_copy(pm_hbm, pm_v)
    pltpu.sync_copy(prb_hbm, prb_v)
    pltpu.sync_copy(tpts_ref.at[:, pl.ds(n0, npt)], coords_v)
    for p in range(NUM_PARTS):
        for ax in range(3):
            pax = 3 * p + ax
            a_b = pa_v[pax, :]
            m_b = pm_v[pax, :]
            rb_b = prb_v[pax, :]

            def chunk_body(c, carry):
                for k in range(CH // 16):
                    x = coords_v[pax, pl.ds(c * CH + k * 16, 16)]
                    gf = jnp.clip((x - m_b) * a_b, 0.0, float(G - 1))
                    g0 = gf.astype(jnp.int32)
                    idx_v[pl.ds(k * 16, 16)] = rb_b + g0
                pltpu.async_copy(tab_ref.at[idx_v], rows_v, sem).wait()
                pltpu.sync_copy(
                    rows_v, out_ref.at[p, ax, pl.ds(n0 + c * CH, CH), :])
                return carry

            lax.fori_loop(0, nchunk, chunk_body, 0)


def _main_body(pid_ref, tpts_ref, view_ref, tflag_ref, dists_ref, tb_ref,
               rows2_ref, feat_ref, w1p_ref, w1v_ref, w1f_ref, b1_ref, w2_ref,
               b2_ref, wsc_ref, raw_ref, occ_ref, occs_ref):
    p = pl.program_id(1)
    B = dists_ref.shape[0]
    dists = dists_ref[...]
    lane128 = jax.lax.broadcasted_iota(jnp.int32, (B, 128), 1)
    lane24 = jax.lax.broadcasted_iota(jnp.int32, (B, 24), 1)
    iota8 = jax.lax.broadcasted_iota(jnp.int32, (B, NUM_PARTS), 1)
    sub24 = jax.lax.broadcasted_iota(jnp.int32, (24, 1), 0)
    r96 = jax.lax.broadcasted_iota(jnp.int32, (128, NC), 0)
    c48 = jax.lax.broadcasted_iota(jnp.int32, (128, NC), 1)
    fold = jnp.logical_or(r96 == c48, r96 == c48 + NC).astype(jnp.bfloat16)

    tpts24 = tpts_ref[...]
    view24 = view_ref[...]
    lines = []
    for ax in range(3):
        sel = (lane24 == 3 * p + ax).astype(jnp.float32)
        x = jnp.sum(tpts24 * sel, axis=1, keepdims=True)
        xmin = tb_ref[0, 0, ax]
        xmax = tb_ref[0, 1, ax]
        gax = jnp.clip((x - xmin) * ((G - 1.0) / (xmax - xmin)), 0.0, G - 1.0)
        tax = gax - jnp.floor(gax)
        vv = rows2_ref[0, ax]
        ww = jnp.where(lane128 < NC, 1.0 - tax,
                       jnp.where(lane128 < 2 * NC, tax, 0.0))
        uu = (vv * ww).astype(jnp.bfloat16)
        lines.append(jnp.dot(uu, fold, preferred_element_type=jnp.float32))

    pmask = jnp.logical_and(sub24 >= 3 * p, sub24 < 3 * p + 3)
    pm = pmask.astype(jnp.float32)
    h_pre = (jnp.dot(tpts24, w1p_ref[...] * pm,
                     preferred_element_type=jnp.float32)
             + jnp.dot(view24, w1v_ref[...] * pm,
                       preferred_element_type=jnp.float32))
    wsc = wsc_ref[p]
    for ip, (i0, i1) in enumerate(MAT_MODE):
        prod = lines[i0] * lines[i1]
        fw = jnp.dot(feat_ref[ip, 0], w1f_ref[0, ip],
                     preferred_element_type=jnp.float32) * wsc
        h_pre = h_pre + jnp.dot(prod, fw, preferred_element_type=jnp.float32)
    h = jnp.maximum(h_pre + b1_ref[0, 0, :], 0.0)
    out = jnp.dot(h, w2_ref[0], preferred_element_type=jnp.float32) + b2_ref[0, 0, :]
    m = jnp.sum(tflag_ref[...] * (iota8 == p).astype(jnp.float32),
                axis=1, keepdims=True)
    raw_p = out[:, :4] * m
    occ_p = (1.0 - jnp.exp(-jnp.maximum(out[:, 4:5], 0.0) * dists)) * m

    @pl.when(p == 0)
    def _():
        raw_ref[...] = jnp.zeros_like(raw_ref)
        occ_ref[...] = jnp.zeros_like(occ_ref)
        occs_ref[...] = jnp.zeros_like(occs_ref)

    raw_ref[...] += raw_p * (1.0 / NUM_PARTS)
    occ_ref[...] += occ_p * (1.0 / NUM_PARTS)
    occs_ref[...] += occ_p * (iota8 == p).astype(jnp.float32)


@jax.jit
def kernel(tpts, viewdir, tflag, dists, part_dist, poses, all_poses, tbounds,
           coord_line, feat_line, W1, b1, W2, b2):
    del part_dist
    N = tpts.shape[0]

    ap3 = all_poses.reshape(POSE_NUM, 24, 3)
    qp3 = poses.reshape(1, 24, 3)
    pid, wscale = pl.pallas_call(
        _pose_select_body,
        out_shape=(
            jax.ShapeDtypeStruct((NUM_PARTS,), jnp.int32),
            jax.ShapeDtypeStruct((NUM_PARTS,), jnp.float32),
        ),
        out_specs=(
            pl.BlockSpec(memory_space=pltpu.SMEM),
            pl.BlockSpec(memory_space=pltpu.SMEM),
        ),
    )(ap3, qp3)

    tt = jnp.swapaxes(coord_line, -1, -2)
    tt_shift = jnp.concatenate([tt[:, :, 1:, :], tt[:, :, -1:, :]], axis=2)
    tab2 = jnp.concatenate([tt, tt_shift], axis=-1).reshape(3 * POSE_NUM * G,
                                                            2 * NC)
    tab2 = jnp.pad(tab2, ((0, 0), (0, 128 - 2 * NC)))

    tpts24 = tpts.reshape(N, 24)
    view24 = viewdir.reshape(N, 24)
    tflag_f = tflag.astype(jnp.float32)
    dists2 = dists.reshape(N, 1)
    W1p = W1[:, 0:3, :].reshape(24, H)
    W1v = W1[:, 3:6, :].reshape(24, H)
    W1f = W1[:, 6:, :].reshape(NUM_PARTS, 3, PD, H)

    tpts_t = tpts24.T
    a_c = (G - 1.0) / (tbounds[0, 1] - tbounds[0, 0])
    pa = jnp.broadcast_to(jnp.tile(a_c, NUM_PARTS)[:, None], (24, 16))
    pm = jnp.broadcast_to(jnp.tile(tbounds[0, 0], NUM_PARTS)[:, None],
                          (24, 16))
    rb = (jnp.repeat(pid, 3) * G
          + jnp.tile(jnp.arange(3, dtype=jnp.int32) * (POSE_NUM * G),
                     NUM_PARTS))
    prb = jnp.broadcast_to(rb[:, None], (24, 16))

    sc_gather = functools.partial(
        pl.kernel,
        mesh=plsc.VectorSubcoreMesh(core_axis_name="c", subcore_axis_name="s"),
        out_type=jax.ShapeDtypeStruct((NUM_PARTS, 3, N, 128), jnp.float32),
        scratch_types=[
            pltpu.VMEM((24, N // NW), jnp.float32),
            pltpu.VMEM((CH,), jnp.int32),
            pltpu.VMEM((CH, 128), jnp.float32),
            pltpu.VMEM((24, 16), jnp.float32),
            pltpu.VMEM((24, 16), jnp.float32),
            pltpu.VMEM((24, 16), jnp.int32),
            pltpu.SemaphoreType.DMA,
        ],
    )(_sc_gather_body)
    rows2 = sc_gather(tab2, tpts_t, pa, pm, prb)

    B = 2048
    grid = (N // B, NUM_PARTS)
    out_shapes = (
        jax.ShapeDtypeStruct((N, 4), jnp.float32),
        jax.ShapeDtypeStruct((N, 1), jnp.float32),
        jax.ShapeDtypeStruct((N, NUM_PARTS), jnp.float32),
    )
    grid_spec = pltpu.PrefetchScalarGridSpec(
        num_scalar_prefetch=1,
        grid=grid,
        in_specs=[
            pl.BlockSpec((B, 24), lambda i, p, pid_ref: (i, 0)),
            pl.BlockSpec((B, 24), lambda i, p, pid_ref: (i, 0)),
            pl.BlockSpec((B, NUM_PARTS), lambda i, p, pid_ref: (i, 0)),
            pl.BlockSpec((B, 1), lambda i, p, pid_ref: (i, 0)),
            pl.BlockSpec((1, 2, 3), lambda i, p, pid_ref: (0, 0, 0)),
            pl.BlockSpec((1, 3, B, 128), lambda i, p, pid_ref: (p, 0, i, 0)),
            pl.BlockSpec((3, 1, NC, PD),
                         lambda i, p, pid_ref: (0, pid_ref[p], 0, 0)),
            pl.BlockSpec((24, H), lambda i, p, pid_ref: (0, 0)),
            pl.BlockSpec((24, H), lambda i, p, pid_ref: (0, 0)),
            pl.BlockSpec((1, 3, PD, H), lambda i, p, pid_ref: (p, 0, 0, 0)),
            pl.BlockSpec((1, 1, H), lambda i, p, pid_ref: (p, 0, 0)),
            pl.BlockSpec((1, H, 5), lambda i, p, pid_ref: (p, 0, 0)),
            pl.BlockSpec((1, 1, 5), lambda i, p, pid_ref: (p, 0, 0)),
            pl.BlockSpec(memory_space=pltpu.SMEM),
        ],
        out_specs=(
            pl.BlockSpec((B, 4), lambda i, p, pid_ref: (i, 0)),
            pl.BlockSpec((B, 1), lambda i, p, pid_ref: (i, 0)),
            pl.BlockSpec((B, NUM_PARTS), lambda i, p, pid_ref: (i, 0)),
        ),
    )
    raw, occ, occs = pl.pallas_call(
        _main_body,
        grid_spec=grid_spec,
        out_shape=out_shapes,
    )(pid, tpts24, view24, tflag_f, dists2, tbounds, rows2, feat_line,
      W1p, W1v, W1f, b1.reshape(NUM_PARTS, 1, H), W2,
      b2.reshape(NUM_PARTS, 1, 5), wscale)
    return raw, occ, occs.reshape(N, NUM_PARTS, 1)


# speedup vs baseline: 1.0300x; 1.0300x over previous
import functools

import jax
import jax.numpy as jnp
from jax import lax
from jax.experimental import pallas as pl
from jax.experimental.pallas import tpu as pltpu
from jax.experimental.pallas import tpu_sc as plsc

NUM_PARTS = 8
POSE_NUM = 16
NC = 48
G = 512
PD = 20
H = 128
MAT_MODE = ((0, 1), (0, 2), (1, 2))
NW = 32
CH = 128


def _pose_select_body(ap_ref, qp_ref, pid_ref, wscale_ref):
    ap = ap_ref[...]
    qp = qp_ref[...]

    def quat(aa):
        angle = jnp.sqrt(jnp.sum(aa * aa, axis=-1))
        half = 0.5 * angle
        small = angle < 1e-6
        sinc = jnp.where(small, 0.5 - angle * angle / 48.0,
                         jnp.sin(half) / jnp.where(small, 1.0, angle))
        return jnp.cos(half), aa * sinc[..., None]

    kw, kxyz = quat(ap)
    qw, qxyz = quat(qp)
    dot = kw * qw + jnp.sum(kxyz * qxyz, axis=-1)
    absdot = jnp.abs(dot)
    iota = jax.lax.broadcasted_iota(jnp.int32, (POSE_NUM, 1), 0)
    for p in range(NUM_PARTS):
        pd = jnp.sum(absdot[:, 3 * p:3 * p + 3], axis=-1, keepdims=True)
        maxv = jnp.max(pd)
        pid = jnp.min(jnp.where(pd == maxv, iota, POSE_NUM))
        pid_ref[p] = pid.astype(jnp.int32)
        wscale_ref[p] = maxv / jnp.maximum(maxv, 1e-16)


def _sc_gather_body(tab_ref, tpts_ref, pa_hbm, pm_hbm, prb_hbm, out_ref,
                    coords_v, idx_v, rows_v, pa_v, pm_v, prb_v, sem):
    n = tpts_ref.shape[1]
    npt = n // NW
    nchunk = npt // CH
    wid = lax.axis_index("s") * 2 + lax.axis_index("c")
    n0 = wid * npt
    pltpu.sync_copy(pa_hbm, pa_v)
    pltpu.sync_copy(pm_hbm, pm_v)
    pltpu.sync_copy(prb_hbm, prb_v)
    pltpu.sync_copy(tpts_ref.at[:, pl.ds(n0, npt)], coords_v)
    for p in range(NUM_PARTS):
        for ax in range(3):
            pax = 3 * p + ax
            a_b = pa_v[pax, :]
            m_b = pm_v[pax, :]
            rb_b = prb_v[pax, :]

            def chunk_body(c, carry):
                for k in range(CH // 16):
                    x = coords_v[pax, pl.ds(c * CH + k * 16, 16)]
                    gf = jnp.clip((x - m_b) * a_b, 0.0, float(G - 1))
                    g0 = gf.astype(jnp.int32)
                    idx_v[pl.ds(k * 16, 16)] = rb_b + g0
                pltpu.async_copy(tab_ref.at[idx_v], rows_v, sem).wait()
                pltpu.sync_copy(
                    rows_v, out_ref.at[p, ax, pl.ds(n0 + c * CH, CH), :])
                return carry

            lax.fori_loop(0, nchunk, chunk_body, 0)


def _main_body(pid_ref, tpts_ref, view_ref, tflag_ref, dists_ref, tb_ref,
               rows2_ref, feat_ref, w1p_ref, w1v_ref, w1f_ref, b1_ref, w2_ref,
               b2_ref, wsc_ref, raw_ref, occ_ref, occs_ref):
    p = pl.program_id(1)
    B = dists_ref.shape[0]
    dists = dists_ref[...]
    lane128 = jax.lax.broadcasted_iota(jnp.int32, (B, 128), 1)
    lane24 = jax.lax.broadcasted_iota(jnp.int32, (B, 24), 1)
    iota8 = jax.lax.broadcasted_iota(jnp.int32, (B, NUM_PARTS), 1)
    sub24 = jax.lax.broadcasted_iota(jnp.int32, (24, 1), 0)
    r96 = jax.lax.broadcasted_iota(jnp.int32, (128, NC), 0)
    c48 = jax.lax.broadcasted_iota(jnp.int32, (128, NC), 1)
    fold = jnp.logical_or(r96 == c48, r96 == c48 + NC).astype(jnp.bfloat16)

    tpts24 = tpts_ref[...]
    view24 = view_ref[...]
    lines = []
    for ax in range(3):
        sel = (lane24 == 3 * p + ax).astype(jnp.float32)
        x = jnp.sum(tpts24 * sel, axis=1, keepdims=True)
        xmin = tb_ref[0, 0, ax]
        xmax = tb_ref[0, 1, ax]
        gax = jnp.clip((x - xmin) * ((G - 1.0) / (xmax - xmin)), 0.0, G - 1.0)
        tax = gax - jnp.floor(gax)
        vv = rows2_ref[0, ax]
        ww = jnp.where(lane128 < NC, 1.0 - tax,
                       jnp.where(lane128 < 2 * NC, tax, 0.0))
        uu = (vv * ww).astype(jnp.bfloat16)
        lines.append(jnp.dot(uu, fold, preferred_element_type=jnp.float32))

    pmask = jnp.logical_and(sub24 >= 3 * p, sub24 < 3 * p + 3)
    pm = pmask.astype(jnp.float32)
    h_pre = (jnp.dot(tpts24, w1p_ref[...] * pm,
                     preferred_element_type=jnp.float32)
             + jnp.dot(view24, w1v_ref[...] * pm,
                       preferred_element_type=jnp.float32))
    wsc = wsc_ref[p]
    for ip, (i0, i1) in enumerate(MAT_MODE):
        prod = lines[i0] * lines[i1]
        fw = jnp.dot(feat_ref[ip, 0], w1f_ref[0, ip],
                     preferred_element_type=jnp.float32) * wsc
        h_pre = h_pre + jnp.dot(prod, fw, preferred_element_type=jnp.float32)
    h = jnp.maximum(h_pre + b1_ref[0, 0, :], 0.0)
    out = jnp.dot(h, w2_ref[0], preferred_element_type=jnp.float32) + b2_ref[0, 0, :]
    m = jnp.sum(tflag_ref[...] * (iota8 == p).astype(jnp.float32),
                axis=1, keepdims=True)
    raw_p = out[:, :4] * m
    occ_p = (1.0 - jnp.exp(-jnp.maximum(out[:, 4:5], 0.0) * dists)) * m

    @pl.when(p == 0)
    def _():
        raw_ref[...] = jnp.zeros_like(raw_ref)
        occ_ref[...] = jnp.zeros_like(occ_ref)
        occs_ref[...] = jnp.zeros_like(occs_ref)

    raw_ref[...] += raw_p * (1.0 / NUM_PARTS)
    occ_ref[...] += occ_p * (1.0 / NUM_PARTS)
    occs_ref[...] += occ_p * (iota8 == p).astype(jnp.float32)


@jax.jit
def kernel(tpts, viewdir, tflag, dists, part_dist, poses, all_poses, tbounds,
           coord_line, feat_line, W1, b1, W2, b2):
    del part_dist
    N = tpts.shape[0]

    ap3 = all_poses.reshape(POSE_NUM, 24, 3)
    qp3 = poses.reshape(1, 24, 3)
    pid, wscale = pl.pallas_call(
        _pose_select_body,
        out_shape=(
            jax.ShapeDtypeStruct((NUM_PARTS,), jnp.int32),
            jax.ShapeDtypeStruct((NUM_PARTS,), jnp.float32),
        ),
        out_specs=(
            pl.BlockSpec(memory_space=pltpu.SMEM),
            pl.BlockSpec(memory_space=pltpu.SMEM),
        ),
    )(ap3, qp3)

    tt = jnp.swapaxes(coord_line, -1, -2)
    tt_shift = jnp.concatenate([tt[:, :, 1:, :], tt[:, :, -1:, :]], axis=2)
    tab2 = jnp.concatenate([tt, tt_shift], axis=-1).reshape(3 * POSE_NUM * G,
                                                            2 * NC)
    tab2 = jnp.pad(tab2, ((0, 0), (0, 128 - 2 * NC)))

    tpts24 = tpts.reshape(N, 24)
    view24 = viewdir.reshape(N, 24)
    tflag_f = tflag.astype(jnp.float32)
    dists2 = dists.reshape(N, 1)
    W1p = W1[:, 0:3, :].reshape(24, H)
    W1v = W1[:, 3:6, :].reshape(24, H)
    W1f = W1[:, 6:, :].reshape(NUM_PARTS, 3, PD, H)

    tpts_t = tpts24.T
    a_c = (G - 1.0) / (tbounds[0, 1] - tbounds[0, 0])
    pa = jnp.broadcast_to(jnp.tile(a_c, NUM_PARTS)[:, None], (24, 16))
    pm = jnp.broadcast_to(jnp.tile(tbounds[0, 0], NUM_PARTS)[:, None],
                          (24, 16))
    rb = (jnp.repeat(pid, 3) * G
          + jnp.tile(jnp.arange(3, dtype=jnp.int32) * (POSE_NUM * G),
                     NUM_PARTS))
    prb = jnp.broadcast_to(rb[:, None], (24, 16))

    sc_gather = functools.partial(
        pl.kernel,
        mesh=plsc.VectorSubcoreMesh(core_axis_name="c", subcore_axis_name="s"),
        out_type=jax.ShapeDtypeStruct((NUM_PARTS, 3, N, 128), jnp.float32),
        scratch_types=[
            pltpu.VMEM((24, N // NW), jnp.float32),
            pltpu.VMEM((CH,), jnp.int32),
            pltpu.VMEM((CH, 128), jnp.float32),
            pltpu.VMEM((24, 16), jnp.float32),
            pltpu.VMEM((24, 16), jnp.float32),
            pltpu.VMEM((24, 16), jnp.int32),
            pltpu.SemaphoreType.DMA,
        ],
    )(_sc_gather_body)
    rows2 = sc_gather(tab2, tpts_t, pa, pm, prb)

    B = 4096
    grid = (N // B, NUM_PARTS)
    out_shapes = (
        jax.ShapeDtypeStruct((N, 4), jnp.float32),
        jax.ShapeDtypeStruct((N, 1), jnp.float32),
        jax.ShapeDtypeStruct((N, NUM_PARTS), jnp.float32),
    )
    grid_spec = pltpu.PrefetchScalarGridSpec(
        num_scalar_prefetch=1,
        grid=grid,
        in_specs=[
            pl.BlockSpec((B, 24), lambda i, p, pid_ref: (i, 0)),
            pl.BlockSpec((B, 24), lambda i, p, pid_ref: (i, 0)),
            pl.BlockSpec((B, NUM_PARTS), lambda i, p, pid_ref: (i, 0)),
            pl.BlockSpec((B, 1), lambda i, p, pid_ref: (i, 0)),
            pl.BlockSpec((1, 2, 3), lambda i, p, pid_ref: (0, 0, 0)),
            pl.BlockSpec((1, 3, B, 128), lambda i, p, pid_ref: (p, 0, i, 0)),
            pl.BlockSpec((3, 1, NC, PD),
                         lambda i, p, pid_ref: (0, pid_ref[p], 0, 0)),
            pl.BlockSpec((24, H), lambda i, p, pid_ref: (0, 0)),
            pl.BlockSpec((24, H), lambda i, p, pid_ref: (0, 0)),
            pl.BlockSpec((1, 3, PD, H), lambda i, p, pid_ref: (p, 0, 0, 0)),
            pl.BlockSpec((1, 1, H), lambda i, p, pid_ref: (p, 0, 0)),
            pl.BlockSpec((1, H, 5), lambda i, p, pid_ref: (p, 0, 0)),
            pl.BlockSpec((1, 1, 5), lambda i, p, pid_ref: (p, 0, 0)),
            pl.BlockSpec(memory_space=pltpu.SMEM),
        ],
        out_specs=(
            pl.BlockSpec((B, 4), lambda i, p, pid_ref: (i, 0)),
            pl.BlockSpec((B, 1), lambda i, p, pid_ref: (i, 0)),
            pl.BlockSpec((B, NUM_PARTS), lambda i, p, pid_ref: (i, 0)),
        ),
    )
    raw, occ, occs = pl.pallas_call(
        _main_body,
        grid_spec=grid_spec,
        out_shape=out_shapes,
    )(pid, tpts24, view24, tflag_f, dists2, tbounds, rows2, feat_line,
      W1p, W1v, W1f, b1.reshape(NUM_PARTS, 1, H), W2,
      b2.reshape(NUM_PARTS, 1, 5), wscale)
    return raw, occ, occs.reshape(N, NUM_PARTS, 1)
